# baseline (speedup 1.0000x reference)
import jax
import jax.numpy as jnp
from jax.experimental import pallas as pl

_TN = 512
_CHUNKS = 8
_GRID = 4


def _vq_kernel(x_ref, v_ref, out_ref):
    v = v_ref[...]
    hb2 = 0.5 * jnp.sum(v * v, axis=-1, keepdims=True)
    def body(j, carry):
        x = x_ref[pl.ds(j * _TN, _TN), :]
        ab = jax.lax.dot_general(
            v, x, (((1,), (1,)), ((), ())),
            preferred_element_type=jnp.float32)
        s = hb2 - ab
        out_ref[j, 0, :] = jnp.argmin(s, axis=0).astype(jnp.int32)
        return carry

    jax.lax.fori_loop(0, _CHUNKS, body, 0, unroll=8)


def kernel(patches, vocab):
    b, n, dim = patches.shape
    k = vocab.shape[0]
    rows = b * n
    x = patches.reshape(rows, dim)
    out = pl.pallas_call(
        _vq_kernel,
        grid=(_GRID,),
        in_specs=[
            pl.BlockSpec((_CHUNKS * _TN, dim), lambda i: (i, 0)),
            pl.BlockSpec((k, dim), lambda i: (0, 0)),
        ],
        out_specs=pl.BlockSpec((_CHUNKS, 1, _TN), lambda i: (i, 0, 0)),
        out_shape=jax.ShapeDtypeStruct((_GRID * _CHUNKS, 1, _TN), jnp.int32),
    )(x, vocab)
    return out.reshape(b, n)


# speedup vs baseline: 1.0674x; 1.0674x over previous
import jax
import jax.numpy as jnp
from jax.experimental import pallas as pl

_TN = 512
_CHUNKS = 16
_GRID = 2
_OUT_COLS = 1024


def _vq_kernel(x_ref, v_ref, out_ref):
    v = v_ref[...]
    hb2 = 0.5 * jnp.sum(v * v, axis=-1, keepdims=True)

    def body(j, carry):
        x = x_ref[pl.ds(j * _TN, _TN), :]
        ab = jax.lax.dot_general(
            v, x, (((1,), (1,)), ((), ())),
            preferred_element_type=jnp.float32)
        s = hb2 - ab
        tok = jnp.argmin(s, axis=0).astype(jnp.int32)
        row = j // (_OUT_COLS // _TN)
        col = (j % (_OUT_COLS // _TN)) * _TN
        out_ref[row, pl.ds(col, _TN)] = tok
        return carry

    jax.lax.fori_loop(0, _CHUNKS, body, 0, unroll=_CHUNKS)


def kernel(patches, vocab):
    b, n, dim = patches.shape
    k = vocab.shape[0]
    rows = b * n
    out_rows_per_step = _CHUNKS * _TN // _OUT_COLS
    x = patches.reshape(rows, dim)
    out = pl.pallas_call(
        _vq_kernel,
        grid=(_GRID,),
        in_specs=[
            pl.BlockSpec((_CHUNKS * _TN, dim), lambda i: (i, 0)),
            pl.BlockSpec((k, dim), lambda i: (0, 0)),
        ],
        out_specs=pl.BlockSpec((out_rows_per_step, _OUT_COLS),
                               lambda i: (i, 0)),
        out_shape=jax.ShapeDtypeStruct((rows // _OUT_COLS, _OUT_COLS),
                                       jnp.int32),
    )(x, vocab)
    return out.reshape(b, n)


# speedup vs baseline: 1.7987x; 1.6851x over previous
import jax
import jax.numpy as jnp
from jax.experimental import pallas as pl

_TN = 512
_GRID = 2
_OUT_COLS = 1024


def _vq_kernel(xt_ref, vt_ref, out_ref):
    vt = vt_ref[...]
    ones = jnp.ones((vt.shape[0], 1), jnp.float32)
    hb2 = 0.5 * jax.lax.dot_general(
        vt * vt, ones, (((0,), (0,)), ((), ())),
        preferred_element_type=jnp.float32)

    nb = xt_ref.shape[0]
    splits = _OUT_COLS // _TN

    def body(j, carry):
        b = j // splits
        c = j % splits
        x = xt_ref[b, :, pl.ds(c * _TN, _TN)]
        ab = jax.lax.dot_general(
            vt, x, (((0,), (0,)), ((), ())),
            preferred_element_type=jnp.float32)
        s = hb2 - ab
        tok = jnp.argmin(s, axis=0).astype(jnp.int32)
        out_ref[b, pl.ds(c * _TN, _TN)] = tok
        return carry

    jax.lax.fori_loop(0, nb * splits, body, 0, unroll=nb * splits)


def kernel(patches, vocab):
    b, n, dim = patches.shape
    k = vocab.shape[0]
    xt = jnp.transpose(patches, (0, 2, 1))
    vt = jnp.transpose(vocab)
    out = pl.pallas_call(
        _vq_kernel,
        grid=(_GRID,),
        in_specs=[
            pl.BlockSpec((b // _GRID, dim, n), lambda i: (i, 0, 0)),
            pl.BlockSpec((dim, k), lambda i: (0, 0)),
        ],
        out_specs=pl.BlockSpec((b // _GRID, n), lambda i: (i, 0)),
        out_shape=jax.ShapeDtypeStruct((b, n), jnp.int32),
    )(xt, vt)
    return out


# speedup vs baseline: 1.8080x; 1.0052x over previous
import jax
import jax.numpy as jnp
from jax.experimental import pallas as pl

_TN = 512
_GRID = 2
_OUT_COLS = 1024


def _vq_kernel(xt_ref, vt_ref, out_ref):
    v = jnp.transpose(vt_ref[...])
    hb2 = 0.5 * jnp.sum(v * v, axis=-1, keepdims=True)

    nb = xt_ref.shape[0]
    splits = _OUT_COLS // _TN

    def body(j, carry):
        b = j // splits
        c = j % splits
        x = xt_ref[b, :, pl.ds(c * _TN, _TN)]
        ab = jax.lax.dot_general(
            v, x, (((1,), (0,)), ((), ())),
            preferred_element_type=jnp.float32)
        s = hb2 - ab
        tok = jnp.argmin(s, axis=0).astype(jnp.int32)
        out_ref[b, pl.ds(c * _TN, _TN)] = tok
        return carry

    jax.lax.fori_loop(0, nb * splits, body, 0, unroll=nb * splits)


def kernel(patches, vocab):
    b, n, dim = patches.shape
    k = vocab.shape[0]
    xt = jnp.transpose(patches, (0, 2, 1))
    vt = jnp.transpose(vocab)
    out = pl.pallas_call(
        _vq_kernel,
        grid=(_GRID,),
        in_specs=[
            pl.BlockSpec((b // _GRID, dim, n), lambda i: (i, 0, 0)),
            pl.BlockSpec((dim, k), lambda i: (0, 0)),
        ],
        out_specs=pl.BlockSpec((b // _GRID, n), lambda i: (i, 0)),
        out_shape=jax.ShapeDtypeStruct((b, n), jnp.int32),
    )(xt, vt)
    return out


# speedup vs baseline: 1.8130x; 1.0028x over previous
import jax
import jax.numpy as jnp
from jax.experimental import pallas as pl
from jax.experimental.pallas import tpu as pltpu

_PW = 256
_NP = 4


def _vq_kernel(xt_hbm, vt_ref, out_ref, xbuf, sem):
    nb = out_ref.shape[0]

    def panel_copy(p, slot):
        return pltpu.make_async_copy(
            xt_hbm.at[:, :, pl.ds(p * _PW, _PW)], xbuf.at[slot], sem.at[slot])

    panel_copy(0, 0).start()
    v = jnp.transpose(vt_ref[...])
    hb2 = 0.5 * jnp.sum(v * v, axis=-1, keepdims=True)

    for p in range(_NP):
        slot = p % 2
        if p + 1 < _NP:
            panel_copy(p + 1, 1 - slot).start()
        panel_copy(p, slot).wait()
        for b in range(nb):
            x = xbuf[slot, b, :, :]
            ab = jax.lax.dot_general(
                v, x, (((1,), (0,)), ((), ())),
                preferred_element_type=jnp.float32)
            s = hb2 - ab
            out_ref[b, pl.ds(p * _PW, _PW)] = (
                jnp.argmin(s, axis=0).astype(jnp.int32))


def kernel(patches, vocab):
    b, n, dim = patches.shape
    k = vocab.shape[0]
    xt = jnp.transpose(patches, (0, 2, 1))
    vt = jnp.transpose(vocab)
    out = pl.pallas_call(
        _vq_kernel,
        in_specs=[
            pl.BlockSpec(memory_space=pltpu.MemorySpace.HBM),
            pl.BlockSpec(memory_space=pltpu.MemorySpace.VMEM),
        ],
        out_specs=pl.BlockSpec(memory_space=pltpu.MemorySpace.VMEM),
        out_shape=jax.ShapeDtypeStruct((b, n), jnp.int32),
        scratch_shapes=[
            pltpu.VMEM((2, b, dim, _PW), jnp.float32),
            pltpu.SemaphoreType.DMA((2,)),
        ],
    )(xt, vt)
    return out


# speedup vs baseline: 1.8151x; 1.0011x over previous
import jax
import jax.numpy as jnp
from jax.experimental import pallas as pl
from jax.experimental.pallas import tpu as pltpu

_PW = 256
_NP = 4


def _vq_kernel(xt_ref, vt_ref, out_ref):
    nb = out_ref.shape[0]
    v = jnp.transpose(vt_ref[...])
    hb2 = 0.5 * jnp.sum(v * v, axis=-1, keepdims=True)

    for p in range(_NP):
        for b in range(nb):
            x = xt_ref[b, :, pl.ds(p * _PW, _PW)]
            ab = jax.lax.dot_general(
                v, x, (((1,), (0,)), ((), ())),
                preferred_element_type=jnp.float32)
            s = hb2 - ab
            out_ref[b, pl.ds(p * _PW, _PW)] = (
                jnp.argmin(s, axis=0).astype(jnp.int32))


def kernel(patches, vocab):
    b, n, dim = patches.shape
    k = vocab.shape[0]
    xt = jnp.transpose(patches, (0, 2, 1))
    vt = jnp.transpose(vocab)
    out = pl.pallas_call(
        _vq_kernel,
        in_specs=[
            pl.BlockSpec(memory_space=pltpu.MemorySpace.VMEM),
            pl.BlockSpec(memory_space=pltpu.MemorySpace.VMEM),
        ],
        out_specs=pl.BlockSpec(memory_space=pltpu.MemorySpace.VMEM),
        out_shape=jax.ShapeDtypeStruct((b, n), jnp.int32),
    )(xt, vt)
    return out
